# baseline (speedup 1.0000x reference)
import jax
import jax.numpy as jnp
from jax import lax
from jax.experimental import pallas as pl
from jax.experimental.pallas import tpu as pltpu
from jax.experimental.pallas import tpu_sc as plsc

ROWS = 8192
COLS = 512
LANES = 16
NSEG = 8
RBLK = 1024


def _rowsum_body(x_ref, o_ref):
    o_ref[...] = jnp.sum(x_ref[...], axis=1)


@jax.jit
def _rowsums(x):
    return pl.pallas_call(
        _rowsum_body,
        grid=(ROWS // RBLK,),
        in_specs=[pl.BlockSpec((RBLK, COLS), lambda i: (i, 0))],
        out_specs=pl.BlockSpec((RBLK,), lambda i: (i,)),
        out_shape=jax.ShapeDtypeStruct((ROWS,), jnp.float32),
    )(x)


def _segprod_body(r_hbm, s_hbm, o_hbm, rbuf, sbuf, obuf):
    cid = lax.axis_index("c")
    sid = lax.axis_index("s")

    @pl.when(jnp.logical_and(cid == 0, sid == 0))
    def _():
        pltpu.sync_copy(s_hbm, sbuf)
        pltpu.sync_copy(r_hbm, rbuf)
        svec = sbuf[...]
        lane = lax.iota(jnp.int32, 16)
        zero = jnp.zeros((LANES,), jnp.float32)
        res = jnp.float32(1.0)
        for i in range(NSEG):
            a = svec[i]
            b = svec[i + 1]
            v0 = lax.div(a, LANES)
            v1 = lax.div(b + (LANES - 1), LANES)

            def body(v, acc, a=a, b=b):
                base = v * LANES
                vec = rbuf[pl.ds(base, LANES)]
                idx = base + lane
                m = (idx >= a) & (idx < b)
                return acc + jnp.where(m, vec, 0.0)

            acc = lax.fori_loop(v0, v1, body, zero)
            p = [acc[j] for j in range(LANES)]
            while len(p) > 1:
                p = [p[j] + p[j + 1] for j in range(0, len(p), 2)]
            res = res * p[0]
        obuf[...] = jnp.broadcast_to(res, (LANES,))
        pltpu.sync_copy(obuf, o_hbm)


@jax.jit
def _segprod(rowsums, s16):
    mesh = plsc.VectorSubcoreMesh(
        core_axis_name="c", subcore_axis_name="s", num_cores=1,
        num_subcores=1)
    f = pl.kernel(
        _segprod_body,
        out_type=jax.ShapeDtypeStruct((LANES,), jnp.float32),
        mesh=mesh,
        scratch_types=[
            pltpu.VMEM((ROWS,), jnp.float32),
            pltpu.VMEM((LANES,), jnp.int32),
            pltpu.VMEM((LANES,), jnp.float32),
        ],
    )
    return f(rowsums, s16)


def kernel(x, slices):
    s16 = jnp.pad(slices.astype(jnp.int32), (0, 7))
    rowsums = _rowsums(x)
    out = _segprod(rowsums, s16)
    return out[0]
